# baseline (speedup 1.0000x reference)
import functools

import jax
import jax.numpy as jnp
from jax import lax
from jax.experimental import pallas as pl
from jax.experimental.pallas import tpu as pltpu
from jax.experimental.pallas import tpu_sc as plsc

_EMBED = 64
_BATCH = 16384
_CHUNK = 128


def _build():
    info = plsc.get_sparse_core_info()
    nc, ns, nl = info.num_cores, info.num_subcores, info.num_lanes
    nw = nc * ns
    b_per_w = _BATCH // nw
    n_chunks = b_per_w // _CHUNK
    mesh = plsc.VectorSubcoreMesh(core_axis_name="c", subcore_axis_name="s")

    @functools.partial(
        pl.kernel,
        mesh=mesh,
        compiler_params=pltpu.CompilerParams(use_tc_tiling_on_sc=False),
        out_type=jax.ShapeDtypeStruct((_BATCH, _EMBED), jnp.float32),
        scratch_types=[
            pltpu.VMEM((n_chunks, _CHUNK), jnp.int32),
            pltpu.VMEM((n_chunks, _CHUNK), jnp.int32),
            pltpu.VMEM((b_per_w, _EMBED), jnp.float32),
            pltpu.VMEM((b_per_w, _EMBED), jnp.float32),
            pltpu.SemaphoreType.DMA,
            pltpu.SemaphoreType.DMA,
        ],
    )
    def gmf(uidx_hbm, iidx_hbm, utab_hbm, itab_hbm, out_hbm,
            uidx_v, iidx_v, urows_v, irows_v, idx_sem, row_sem):
        wid = lax.axis_index("s") * nc + lax.axis_index("c")
        row0 = wid * n_chunks
        cu = pltpu.async_copy(uidx_hbm.at[pl.ds(row0, n_chunks)], uidx_v, idx_sem)
        ci = pltpu.async_copy(iidx_hbm.at[pl.ds(row0, n_chunks)], iidx_v, idx_sem)
        cu.wait()
        ci.wait()

        gathers = []
        for k in range(n_chunks):
            sl = pl.ds(k * _CHUNK, _CHUNK)
            gathers.append(
                pltpu.async_copy(utab_hbm.at[uidx_v.at[k]], urows_v.at[sl], row_sem))
            gathers.append(
                pltpu.async_copy(itab_hbm.at[iidx_v.at[k]], irows_v.at[sl], row_sem))
        for g in gathers:
            g.wait()

        def body(r, carry):
            for j in range(_EMBED // nl):
                s = pl.ds(j * nl, nl)
                urows_v[r, s] = urows_v[r, s] * irows_v[r, s]
            return carry
        lax.fori_loop(0, b_per_w, body, 0)

        pltpu.sync_copy(urows_v, out_hbm.at[pl.ds(wid * b_per_w, b_per_w)])

    return gmf


_gmf = _build()


def kernel(user_indices, item_indices, user_table, item_table):
    u_idx = user_indices.astype(jnp.int32).reshape(_BATCH // _CHUNK, _CHUNK)
    i_idx = item_indices.astype(jnp.int32).reshape(_BATCH // _CHUNK, _CHUNK)
    return _gmf(u_idx, i_idx, user_table, item_table)


# speedup vs baseline: 2.1815x; 2.1815x over previous
import functools

import jax
import jax.numpy as jnp
from jax import lax
from jax.experimental import pallas as pl
from jax.experimental.pallas import tpu as pltpu
from jax.experimental.pallas import tpu_sc as plsc

_EMBED = 64
_BATCH = 16384
_ROWS = 1000000
_CHUNK = 16
_NCHUNK = None


def _build():
    info = plsc.get_sparse_core_info()
    nc, ns, nl = info.num_cores, info.num_subcores, info.num_lanes
    nw = nc * ns
    b_per_w = _BATCH // nw
    n_chunks = b_per_w // _CHUNK
    idx_rows_per_w = b_per_w // 128
    mesh = plsc.VectorSubcoreMesh(core_axis_name="c", subcore_axis_name="s")

    @functools.partial(
        pl.kernel,
        mesh=mesh,
        out_type=jax.ShapeDtypeStruct((_BATCH, _EMBED), jnp.float32),
        scratch_types=[
            pltpu.VMEM((idx_rows_per_w, 128), jnp.int32),
            pltpu.VMEM((idx_rows_per_w, 128), jnp.int32),
            pltpu.VMEM((idx_rows_per_w, 128), jnp.int32),
            pltpu.VMEM((idx_rows_per_w, 128), jnp.int32),
            pltpu.VMEM((_CHUNK, 8, _EMBED), jnp.float32),
            pltpu.VMEM((_CHUNK, 8, _EMBED), jnp.float32),
            pltpu.VMEM((_CHUNK, 8, _EMBED), jnp.float32),
            pltpu.VMEM((_CHUNK, 8, _EMBED), jnp.float32),
            pltpu.VMEM((_CHUNK, _EMBED), jnp.float32),
            pltpu.VMEM((_CHUNK, _EMBED), jnp.float32),
            pltpu.SemaphoreType.DMA,
            pltpu.SemaphoreType.DMA,
            pltpu.SemaphoreType.DMA,
            pltpu.SemaphoreType.DMA,
        ],
    )
    def gmf(ublk_hbm, usub_hbm, iblk_hbm, isub_hbm, utab_hbm, itab_hbm, out_hbm,
            ublk_v, usub_v, iblk_v, isub_v,
            ub0, ub1, ib0, ib1, prod0, prod1,
            gsem0, gsem1, osem0, osem1):
        wid = lax.axis_index("s") * nc + lax.axis_index("c")
        base = wid * b_per_w
        irow0 = wid * idx_rows_per_w
        pltpu.sync_copy(ublk_hbm.at[pl.ds(irow0, idx_rows_per_w)], ublk_v)
        pltpu.sync_copy(usub_hbm.at[pl.ds(irow0, idx_rows_per_w)], usub_v)
        pltpu.sync_copy(iblk_hbm.at[pl.ds(irow0, idx_rows_per_w)], iblk_v)
        pltpu.sync_copy(isub_hbm.at[pl.ds(irow0, idx_rows_per_w)], isub_v)

        ub = (ub0, ub1)
        ib = (ib0, ib1)
        prod = (prod0, prod1)
        gsem = (gsem0, gsem1)
        osem = (osem0, osem1)

        def issue(c, parity):
            r = lax.shift_right_logical(c, 3)
            o = lax.mul(lax.bitwise_and(c, 7), 16)
            ublkv = ublk_v[r, pl.ds(o, 16)]
            iblkv = iblk_v[r, pl.ds(o, 16)]
            for i in range(_CHUNK):
                pltpu.async_copy(utab_hbm.at[ublkv[i]], ub[parity].at[i],
                                 gsem[parity])
                pltpu.async_copy(itab_hbm.at[iblkv[i]], ib[parity].at[i],
                                 gsem[parity])

        issue(jnp.int32(0), 0)

        def body(k, carry):
            for b in (0, 1):
                c = 2 * k + b
                bn = (b + 1) & 1
                cn = c + 1

                @pl.when(cn < n_chunks)
                def _():
                    issue(cn, bn)

                pltpu.make_async_copy(utab_hbm.at[pl.ds(0, _CHUNK)], ub[b],
                                      gsem[b]).wait()
                pltpu.make_async_copy(itab_hbm.at[pl.ds(0, _CHUNK)], ib[b],
                                      gsem[b]).wait()

                @pl.when(c >= 2)
                def _():
                    pltpu.make_async_copy(out_hbm.at[pl.ds(0, _CHUNK)],
                                          prod[b], osem[b]).wait()

                r = lax.shift_right_logical(c, 3)
                o = lax.mul(lax.bitwise_and(c, 7), 16)
                suv = usub_v[r, pl.ds(o, 16)]
                siv = isub_v[r, pl.ds(o, 16)]
                for i in range(_CHUNK):
                    su = suv[i]
                    si = siv[i]
                    for j in range(_EMBED // nl):
                        s = pl.ds(j * nl, nl)
                        prod[b][i, s] = ub[b][i, su, s] * ib[b][i, si, s]

                pltpu.async_copy(prod[b], out_hbm.at[pl.ds(base + c * _CHUNK,
                                                           _CHUNK)], osem[b])
            return carry

        lax.fori_loop(0, n_chunks // 2, body, 0)

        pltpu.make_async_copy(out_hbm.at[pl.ds(0, _CHUNK)], prod0, osem0).wait()
        pltpu.make_async_copy(out_hbm.at[pl.ds(0, _CHUNK)], prod1, osem1).wait()

    return gmf


_gmf = _build()


def kernel(user_indices, item_indices, user_table, item_table):
    uidx = user_indices.astype(jnp.int32)
    iidx = item_indices.astype(jnp.int32)
    ublk = (uidx >> 3).reshape(128, 128)
    usub = (uidx & 7).reshape(128, 128)
    iblk = (iidx >> 3).reshape(128, 128)
    isub = (iidx & 7).reshape(128, 128)
    utab3 = user_table.reshape(_ROWS // 8, 8, _EMBED)
    itab3 = item_table.reshape(_ROWS // 8, 8, _EMBED)
    return _gmf(ublk, usub, iblk, isub, utab3, itab3)
